# Initial kernel scaffold; baseline (speedup 1.0000x reference)
import jax
import jax.numpy as jnp
from jax.experimental import pallas as pl


def kernel(X, edge_index, batch, Wl, Wr, b):
    raise NotImplementedError("write your pallas kernel here")


# speedup vs baseline: 4.3563x; 4.3563x over previous
import functools

import jax
import jax.numpy as jnp
from jax import lax
from jax.experimental import pallas as pl
from jax.experimental.pallas import tpu as pltpu
from jax.experimental.pallas import tpu_sc as plsc

N = 10000
E = 320000
D = 128
G = 64

NC = 2
NS = 16
NW = NC * NS
EPT = E // NW
EK = 80
NITER = EPT // EK
RPT = 624
TBASE = NS * RPT
TAIL = N - TBASE
DEGW = 16

BN = 1024
NG = (N + BN - 1) // BN


def _sc_mesh():
    return plsc.VectorSubcoreMesh(core_axis_name="c", subcore_axis_name="s")


@functools.partial(
    pl.kernel,
    mesh=_sc_mesh(),
    out_type=jax.ShapeDtypeStruct((NC * N, D), jnp.float32),
    scratch_types=[
        pltpu.VMEM((EK,), jnp.int32),
        pltpu.VMEM((EK,), jnp.int32),
        pltpu.VMEM((EK, D), jnp.float32),
        pltpu.VMEM_SHARED((N, D), jnp.float32),
        pltpu.SemaphoreType.DMA,
    ],
)
def _sc_agg(p_hbm, src_hbm, dst_hbm, zero_hbm, out_hbm,
            idx_s, idx_d, rows, acc, sem):
    c = lax.axis_index("c")
    s = lax.axis_index("s")
    wid = s * NC + c
    r0 = s * RPT
    pltpu.sync_copy(zero_hbm.at[pl.ds(r0, RPT)], acc.at[pl.ds(r0, RPT)])

    @pl.when(s == NS - 1)
    def _():
        pltpu.sync_copy(zero_hbm.at[pl.ds(TBASE, TAIL)],
                        acc.at[pl.ds(TBASE, TAIL)])

    plsc.subcore_barrier()

    base = wid * EPT

    def body(j, carry):
        e0 = base + j * EK
        pltpu.sync_copy(src_hbm.at[pl.ds(e0, EK)], idx_s)
        pltpu.sync_copy(dst_hbm.at[pl.ds(e0, EK)], idx_d)
        pltpu.async_copy(p_hbm.at[idx_s], rows, sem).wait()
        pltpu.sync_copy(rows, acc.at[idx_d], add=True)
        return carry

    lax.fori_loop(0, NITER, body, 0)
    plsc.subcore_barrier()
    pltpu.sync_copy(acc.at[pl.ds(r0, RPT)], out_hbm.at[pl.ds(c * N + r0, RPT)])

    @pl.when(s == NS - 1)
    def _():
        pltpu.sync_copy(acc.at[pl.ds(TBASE, TAIL)],
                        out_hbm.at[pl.ds(c * N + TBASE, TAIL)])


@functools.partial(
    pl.kernel,
    mesh=_sc_mesh(),
    out_type=jax.ShapeDtypeStruct((NC * N, D), jnp.float32),
    scratch_types=[
        pltpu.VMEM((EK,), jnp.int32),
        pltpu.VMEM((EK, D), jnp.float32),
        pltpu.VMEM_SHARED((N, D), jnp.float32),
    ],
)
def _sc_deg(dst_hbm, ones_hbm, zero_hbm, out_hbm, idx_d, ones_v, acc):
    c = lax.axis_index("c")
    s = lax.axis_index("s")
    wid = s * NC + c
    r0 = s * RPT
    pltpu.sync_copy(zero_hbm.at[pl.ds(r0, RPT)], acc.at[pl.ds(r0, RPT)])
    pltpu.sync_copy(ones_hbm, ones_v)

    @pl.when(s == NS - 1)
    def _():
        pltpu.sync_copy(zero_hbm.at[pl.ds(TBASE, TAIL)],
                        acc.at[pl.ds(TBASE, TAIL)])

    plsc.subcore_barrier()

    base = wid * EPT

    def body(j, carry):
        e0 = base + j * EK
        pltpu.sync_copy(dst_hbm.at[pl.ds(e0, EK)], idx_d)
        pltpu.sync_copy(ones_v, acc.at[idx_d], add=True)
        return carry

    lax.fori_loop(0, NITER, body, 0)
    plsc.subcore_barrier()
    pltpu.sync_copy(acc.at[pl.ds(r0, RPT)], out_hbm.at[pl.ds(c * N + r0, RPT)])

    @pl.when(s == NS - 1)
    def _():
        pltpu.sync_copy(acc.at[pl.ds(TBASE, TAIL)],
                        out_hbm.at[pl.ds(c * N + TBASE, TAIL)])


def _leaky(x):
    return jnp.where(x >= 0, x, 0.01 * x)


def _tc0_body(x_ref, deg_ref, wl_ref, wr_ref, b_ref, p_ref, r_ref, inv_ref):
    h = x_ref[...]
    p_ref[...] = jnp.dot(h, wl_ref[...], preferred_element_type=jnp.float32)
    r_ref[...] = jnp.dot(h, wr_ref[...], preferred_element_type=jnp.float32) + b_ref[...]
    dsum = deg_ref[0] + deg_ref[1]
    inv = 1.0 / jnp.maximum(dsum[:, 0:1], 1.0)
    inv_ref[...] = jnp.broadcast_to(inv, (BN, D))


def _tc0(x, deg2, wl, wr, b):
    return pl.pallas_call(
        _tc0_body,
        grid=(NG,),
        in_specs=[
            pl.BlockSpec((BN, D), lambda i: (i, 0)),
            pl.BlockSpec((2, BN, D), lambda i: (0, i, 0)),
            pl.BlockSpec((D, D), lambda i: (0, 0)),
            pl.BlockSpec((D, D), lambda i: (0, 0)),
            pl.BlockSpec((1, D), lambda i: (0, 0)),
        ],
        out_specs=[pl.BlockSpec((BN, D), lambda i: (i, 0))] * 3,
        out_shape=[jax.ShapeDtypeStruct((N, D), jnp.float32)] * 3,
    )(x, deg2, wl, wr, b)


def _combine_body_res(agg_ref, inv_ref, rp_ref, hp_ref, wl_ref, wr_ref, b_ref,
                      h_ref, p_ref, r_ref):
    t = (agg_ref[0] + agg_ref[1]) * inv_ref[...] + rp_ref[...]
    h = hp_ref[...] + _leaky(t)
    h_ref[...] = h
    p_ref[...] = jnp.dot(h, wl_ref[...], preferred_element_type=jnp.float32)
    r_ref[...] = jnp.dot(h, wr_ref[...], preferred_element_type=jnp.float32) + b_ref[...]


def _combine_body_nores(agg_ref, inv_ref, rp_ref, wl_ref, wr_ref, b_ref,
                        h_ref, p_ref, r_ref):
    t = (agg_ref[0] + agg_ref[1]) * inv_ref[...] + rp_ref[...]
    h = _leaky(t)
    h_ref[...] = h
    p_ref[...] = jnp.dot(h, wl_ref[...], preferred_element_type=jnp.float32)
    r_ref[...] = jnp.dot(h, wr_ref[...], preferred_element_type=jnp.float32) + b_ref[...]


def _tc_combine(agg2, invdeg, r_prev, h_prev, wl, wr, b, residual):
    nd_spec = pl.BlockSpec((BN, D), lambda i: (i, 0))
    in_specs = [
        pl.BlockSpec((2, BN, D), lambda i: (0, i, 0)),
        nd_spec,
        nd_spec,
    ]
    args = [agg2, invdeg, r_prev]
    if residual:
        in_specs.append(nd_spec)
        args.append(h_prev)
    in_specs += [
        pl.BlockSpec((D, D), lambda i: (0, 0)),
        pl.BlockSpec((D, D), lambda i: (0, 0)),
        pl.BlockSpec((1, D), lambda i: (0, 0)),
    ]
    args += [wl, wr, b]
    body = _combine_body_res if residual else _combine_body_nores
    return pl.pallas_call(
        body,
        grid=(NG,),
        in_specs=in_specs,
        out_specs=[nd_spec] * 3,
        out_shape=[jax.ShapeDtypeStruct((N, D), jnp.float32)] * 3,
    )(*args)


def _pool_body(agg_ref, inv_ref, rp_ref, hp_ref, batch_ref, out_ref, cnt_ref):
    gi = pl.program_id(0)
    t = (agg_ref[0] + agg_ref[1]) * inv_ref[...] + rp_ref[...]
    h4 = hp_ref[...] + _leaky(t)

    bt = batch_ref[...]
    gid = lax.broadcasted_iota(jnp.int32, (G, BN), 0)
    col = lax.broadcasted_iota(jnp.int32, (G, BN), 1) + gi * BN
    m = jnp.logical_and(bt == gid, col < N)
    mf = m.astype(jnp.float32)

    @pl.when(gi == 0)
    def _():
        out_ref[...] = jnp.zeros((G, D), jnp.float32)
        cnt_ref[...] = jnp.zeros((G, D), jnp.float32)

    out_ref[...] += jnp.dot(mf, h4, preferred_element_type=jnp.float32)
    cnt_ref[...] += jnp.broadcast_to(
        jnp.sum(mf, axis=1, keepdims=True), (G, D))

    @pl.when(gi == NG - 1)
    def _():
        out_ref[...] = out_ref[...] / jnp.maximum(cnt_ref[...], 1.0)


def _tc_pool(agg2, invdeg, r_prev, h_prev, batch2):
    nd_spec = pl.BlockSpec((BN, D), lambda i: (i, 0))
    return pl.pallas_call(
        _pool_body,
        grid=(NG,),
        in_specs=[
            pl.BlockSpec((2, BN, D), lambda i: (0, i, 0)),
            nd_spec,
            nd_spec,
            nd_spec,
            pl.BlockSpec((1, BN), lambda i: (0, i)),
        ],
        out_specs=pl.BlockSpec((G, D), lambda i: (0, 0)),
        out_shape=jax.ShapeDtypeStruct((G, D), jnp.float32),
        scratch_shapes=[pltpu.VMEM((G, D), jnp.float32)],
    )(agg2, invdeg, r_prev, h_prev, batch2)


def kernel(X, edge_index, batch, Wl, Wr, b):
    src = edge_index[0]
    dst = edge_index[1]
    batch2 = batch.reshape(1, N)
    b2 = b.reshape(4, 1, D)

    zeros_nd = jnp.zeros((N, D), jnp.float32)
    ones_ek = jnp.ones((EK, D), jnp.float32)

    deg2 = _sc_deg(dst, ones_ek, zeros_nd).reshape(2, N, D)

    p0, r0, invdeg = _tc0(X, deg2, Wl[0], Wr[0], b2[0])

    agg0 = _sc_agg(p0, src, dst, zeros_nd).reshape(2, N, D)
    h1, p1, r1 = _tc_combine(agg0, invdeg, r0, None, Wl[1], Wr[1], b2[1],
                             residual=False)

    agg1 = _sc_agg(p1, src, dst, zeros_nd).reshape(2, N, D)
    h2, p2, r2 = _tc_combine(agg1, invdeg, r1, None, Wl[2], Wr[2], b2[2],
                             residual=False)

    agg2_ = _sc_agg(p2, src, dst, zeros_nd).reshape(2, N, D)
    h3, p3, r3 = _tc_combine(agg2_, invdeg, r2, h2, Wl[3], Wr[3], b2[3],
                             residual=True)

    agg3 = _sc_agg(p3, src, dst, zeros_nd).reshape(2, N, D)
    return _tc_pool(agg3, invdeg, r3, h3, batch2)
